# Initial kernel scaffold; baseline (speedup 1.0000x reference)
import jax
import jax.numpy as jnp
from jax.experimental import pallas as pl


def kernel(x, edge_index, edge_attr, batch, Wl1, bl1, Wr1, Wl2, bl2, Wr2):
    raise NotImplementedError("write your pallas kernel here")


# speedup vs baseline: 4.8338x; 4.8338x over previous
import functools

import jax
import jax.numpy as jnp
from jax import lax
from jax.experimental import pallas as pl
from jax.experimental.pallas import tpu as pltpu
from jax.experimental.pallas import tpu_sc as plsc

N = 10000
E = 320000
D = 128
G = 64
DW = 128
NC = 2
NS = 16
NW = NC * NS
CH = 80
EW = E // NW
NCH = EW // CH
NRCH = N // CH


@functools.lru_cache(maxsize=None)
def _make_sc_agg():
    mesh = plsc.VectorSubcoreMesh(core_axis_name="c", subcore_axis_name="s")

    @functools.partial(
        pl.kernel, mesh=mesh,
        out_type=[jax.ShapeDtypeStruct((NC, N, D), jnp.float32)],
        scratch_types=[
            pltpu.VMEM((CH,), jnp.int32),
            pltpu.VMEM((CH,), jnp.int32),
            pltpu.VMEM((CH, D), jnp.float32),
            pltpu.VMEM_SHARED((N, D), jnp.float32),
            pltpu.SemaphoreType.DMA,
        ],
    )
    def sc_agg(table, srci, dsti, z2, outp, sidx, didx, rows, acc, sem):
        cid = lax.axis_index("c")
        sid = lax.axis_index("s")
        wid = sid * NC + cid

        pltpu.sync_copy(z2, rows)

        def zero_chunk(k, c):
            m = sid * 8 + k

            @pl.when(m < NRCH)
            def _():
                pltpu.sync_copy(rows, acc.at[pl.ds(m * CH, CH)])

            return c

        lax.fori_loop(0, 8, zero_chunk, 0)
        plsc.subcore_barrier()

        base = wid * EW

        def step(j, carry):
            e0 = base + j * CH
            pltpu.sync_copy(srci.at[pl.ds(e0, CH)], sidx)
            pltpu.sync_copy(dsti.at[pl.ds(e0, CH)], didx)
            pltpu.async_copy(table.at[sidx], rows, sem).wait()
            pltpu.sync_copy(rows, acc.at[didx], add=True)
            return carry

        lax.fori_loop(0, NCH, step, 0)
        plsc.subcore_barrier()

        def pub_chunk(k, c):
            m = sid * 8 + k

            @pl.when(m < NRCH)
            def _():
                r0 = m * CH
                pltpu.sync_copy(acc.at[pl.ds(r0, CH)], rows)
                pltpu.sync_copy(rows, outp.at[cid, pl.ds(r0, CH)])

            return c

        lax.fori_loop(0, 8, pub_chunk, 0)

    return sc_agg


@functools.lru_cache(maxsize=None)
def _make_sc_deg():
    mesh = plsc.VectorSubcoreMesh(core_axis_name="c", subcore_axis_name="s")

    @functools.partial(
        pl.kernel, mesh=mesh,
        out_type=[jax.ShapeDtypeStruct((NC, N, DW), jnp.float32)],
        scratch_types=[
            pltpu.VMEM((CH,), jnp.int32),
            pltpu.VMEM((CH, DW), jnp.float32),
            pltpu.VMEM_SHARED((N, DW), jnp.float32),
        ],
    )
    def sc_deg(dsti, z1, o1, outdeg, didx, ones_v, deg_sh):
        cid = lax.axis_index("c")
        sid = lax.axis_index("s")
        wid = sid * NC + cid

        pltpu.sync_copy(z1, ones_v)

        def zero_chunk(k, c):
            m = sid * 8 + k

            @pl.when(m < NRCH)
            def _():
                pltpu.sync_copy(ones_v, deg_sh.at[pl.ds(m * CH, CH)])

            return c

        lax.fori_loop(0, 8, zero_chunk, 0)
        pltpu.sync_copy(o1, ones_v)
        plsc.subcore_barrier()

        base = wid * EW

        def step(j, carry):
            e0 = base + j * CH
            pltpu.sync_copy(dsti.at[pl.ds(e0, CH)], didx)
            pltpu.sync_copy(ones_v, deg_sh.at[didx], add=True)
            return carry

        lax.fori_loop(0, NCH, step, 0)
        plsc.subcore_barrier()

        def pub_chunk(k, c):
            m = sid * 8 + k

            @pl.when(m < NRCH)
            def _():
                r0 = m * CH
                pltpu.sync_copy(deg_sh.at[pl.ds(r0, CH)], ones_v)
                pltpu.sync_copy(ones_v, outdeg.at[cid, pl.ds(r0, CH)])

            return c

        lax.fori_loop(0, 8, pub_chunk, 0)

    return sc_deg


def _tc_in2(x, Wl, Wr, b):
    def body(x_ref, wl_ref, wr_ref, b_ref, yl_ref, yr_ref):
        xv = x_ref[...]
        yl_ref[...] = jnp.dot(xv, wl_ref[...], preferred_element_type=jnp.float32)
        yr_ref[...] = jnp.dot(xv, wr_ref[...], preferred_element_type=jnp.float32) + b_ref[...]

    return pl.pallas_call(
        body,
        out_shape=(jax.ShapeDtypeStruct((N, D), jnp.float32),
                   jax.ShapeDtypeStruct((N, D), jnp.float32)),
    )(x, Wl, Wr, b)


def _tc_mid(p, degp, y1r, Wl2, Wr2, b2):
    blk = 1000

    def body(p_ref, dp_ref, y1r_ref, wl_ref, wr_ref, b_ref, yl_ref, yr_ref):
        dp = dp_ref[...]
        rinv = 1.0 / jnp.maximum(dp[0, :, 0:1] + dp[1, :, 0:1], 1.0)
        pv = p_ref[...]
        h = jnp.maximum((pv[0] + pv[1]) * rinv + y1r_ref[...], 0.0)
        yl_ref[...] = jnp.dot(h, wl_ref[...], preferred_element_type=jnp.float32)
        yr_ref[...] = jnp.dot(h, wr_ref[...], preferred_element_type=jnp.float32) + b_ref[...]

    return pl.pallas_call(
        body,
        grid=(N // blk,),
        in_specs=[
            pl.BlockSpec((NC, blk, D), lambda i: (0, i, 0)),
            pl.BlockSpec((NC, blk, DW), lambda i: (0, i, 0)),
            pl.BlockSpec((blk, D), lambda i: (i, 0)),
            pl.BlockSpec((D, D), lambda i: (0, 0)),
            pl.BlockSpec((D, D), lambda i: (0, 0)),
            pl.BlockSpec((1, D), lambda i: (0, 0)),
        ],
        out_specs=(pl.BlockSpec((blk, D), lambda i: (i, 0)),
                   pl.BlockSpec((blk, D), lambda i: (i, 0))),
        out_shape=(jax.ShapeDtypeStruct((N, D), jnp.float32),
                   jax.ShapeDtypeStruct((N, D), jnp.float32)),
    )(p, degp, y1r, Wl2, Wr2, b2)


def _tc_pool(p, degp, y2r, batch2):
    def body(p_ref, dp_ref, y2r_ref, b_ref, out_ref):
        dp = dp_ref[...]
        rinv = 1.0 / jnp.maximum(dp[0, :, 0:1] + dp[1, :, 0:1], 1.0)
        pv = p_ref[...]
        nodes = (pv[0] + pv[1]) * rinv + y2r_ref[...]
        gids = lax.broadcasted_iota(jnp.int32, (G, N), 0)
        m = (b_ref[...] == gids).astype(jnp.float32)
        s = jnp.dot(m, nodes, preferred_element_type=jnp.float32)
        cnt = jnp.sum(m, axis=1, keepdims=True)
        out_ref[...] = s / jnp.maximum(cnt, 1.0)

    return pl.pallas_call(
        body,
        out_shape=jax.ShapeDtypeStruct((G, D), jnp.float32),
    )(p, degp, y2r, batch2)


def kernel(x, edge_index, edge_attr, batch, Wl1, bl1, Wr1, Wl2, bl2, Wr2):
    src = edge_index[0]
    dst = edge_index[1]
    z2 = jnp.zeros((CH, D), jnp.float32)
    z1 = jnp.zeros((CH, DW), jnp.float32)
    o1 = jnp.ones((CH, DW), jnp.float32)
    (degp,) = _make_sc_deg()(dst, z1, o1)
    y1l, y1r = _tc_in2(x, Wl1, Wr1, bl1.reshape(1, D))
    (p1,) = _make_sc_agg()(y1l, src, dst, z2)
    y2l, y2r = _tc_mid(p1, degp, y1r, Wl2, Wr2, bl2.reshape(1, D))
    (p2,) = _make_sc_agg()(y2l, src, dst, z2)
    return _tc_pool(p2, degp, y2r, batch.reshape(1, N))


# speedup vs baseline: 8.6771x; 1.7951x over previous
import functools

import jax
import jax.numpy as jnp
from jax import lax
from jax.experimental import pallas as pl
from jax.experimental.pallas import tpu as pltpu
from jax.experimental.pallas import tpu_sc as plsc

N = 10000
E = 320000
D = 128
G = 64
DW = 128
NC = 2
NS = 16
NW = NC * NS
CH = 80
EW = E // NW
NCH = EW // CH
NRCH = N // CH


@functools.lru_cache(maxsize=None)
def _make_sc_agg():
    mesh = plsc.VectorSubcoreMesh(core_axis_name="c", subcore_axis_name="s")

    @functools.partial(
        pl.kernel, mesh=mesh,
        out_type=[jax.ShapeDtypeStruct((NC, N, D), jnp.float32)],
        scratch_types=[
            pltpu.VMEM((2, CH), jnp.int32),
            pltpu.VMEM((2, CH), jnp.int32),
            pltpu.VMEM((2, CH, D), jnp.float32),
            pltpu.VMEM_SHARED((N, D), jnp.float32),
            pltpu.SemaphoreType.DMA,
            pltpu.SemaphoreType.DMA,
            pltpu.SemaphoreType.DMA,
        ],
    )
    def sc_agg(table, srci, dsti, z2, outp,
               sidx, didx, rows, acc, semi, semg, sems):
        cid = lax.axis_index("c")
        sid = lax.axis_index("s")
        wid = sid * NC + cid

        pltpu.sync_copy(z2, rows.at[0])

        def zero_chunk(k, c):
            m = sid * 8 + k

            @pl.when(m < NRCH)
            def _():
                pltpu.sync_copy(rows.at[0], acc.at[pl.ds(m * CH, CH)])

            return c

        lax.fori_loop(0, 8, zero_chunk, 0)
        plsc.subcore_barrier()

        base = wid * EW

        pltpu.async_copy(srci.at[pl.ds(base, CH)], sidx.at[0], semi)
        pltpu.async_copy(dsti.at[pl.ds(base, CH)], didx.at[0], semi)

        def step(j, carry):
            b = lax.rem(j, 2)

            @pl.when(j >= 2)
            def _():
                pltpu.make_async_copy(
                    rows.at[b], acc.at[pl.ds(0, CH)], sems).wait()

            pltpu.make_async_copy(
                srci.at[pl.ds(base, CH)], sidx.at[b], semi).wait()
            pltpu.make_async_copy(
                dsti.at[pl.ds(base, CH)], didx.at[b], semi).wait()

            @pl.when(j + 1 < NCH)
            def _():
                e1 = base + (j + 1) * CH
                pltpu.async_copy(srci.at[pl.ds(e1, CH)], sidx.at[1 - b], semi)
                pltpu.async_copy(dsti.at[pl.ds(e1, CH)], didx.at[1 - b], semi)

            pltpu.async_copy(table.at[sidx.at[b]], rows.at[b], semg).wait()
            pltpu.async_copy(rows.at[b], acc.at[didx.at[b]], sems, add=True)
            return carry

        lax.fori_loop(0, NCH, step, 0)
        pltpu.make_async_copy(rows.at[1], acc.at[pl.ds(0, CH)], sems).wait()
        pltpu.make_async_copy(rows.at[0], acc.at[pl.ds(0, CH)], sems).wait()
        plsc.subcore_barrier()

        def pub_chunk(k, c):
            m = sid * 8 + k

            @pl.when(m < NRCH)
            def _():
                r0 = m * CH
                pltpu.sync_copy(acc.at[pl.ds(r0, CH)], rows.at[0])
                pltpu.sync_copy(rows.at[0], outp.at[cid, pl.ds(r0, CH)])

            return c

        lax.fori_loop(0, 8, pub_chunk, 0)

    return sc_agg


@functools.lru_cache(maxsize=None)
def _make_sc_deg():
    mesh = plsc.VectorSubcoreMesh(core_axis_name="c", subcore_axis_name="s")

    @functools.partial(
        pl.kernel, mesh=mesh,
        out_type=[jax.ShapeDtypeStruct((NC, N, DW), jnp.float32)],
        scratch_types=[
            pltpu.VMEM((2, CH), jnp.int32),
            pltpu.VMEM((CH, DW), jnp.float32),
            pltpu.VMEM_SHARED((N, DW), jnp.float32),
            pltpu.SemaphoreType.DMA,
            pltpu.SemaphoreType.DMA,
        ],
    )
    def sc_deg(dsti, z1, o1, outdeg, didx, ones_v, deg_sh, semi, sems):
        cid = lax.axis_index("c")
        sid = lax.axis_index("s")
        wid = sid * NC + cid

        pltpu.sync_copy(z1, ones_v)

        def zero_chunk(k, c):
            m = sid * 8 + k

            @pl.when(m < NRCH)
            def _():
                pltpu.sync_copy(ones_v, deg_sh.at[pl.ds(m * CH, CH)])

            return c

        lax.fori_loop(0, 8, zero_chunk, 0)
        pltpu.sync_copy(o1, ones_v)
        plsc.subcore_barrier()

        base = wid * EW

        pltpu.async_copy(dsti.at[pl.ds(base, CH)], didx.at[0], semi)

        def step(j, carry):
            b = lax.rem(j, 2)

            @pl.when(j >= 2)
            def _():
                pltpu.make_async_copy(
                    ones_v, deg_sh.at[pl.ds(0, CH)], sems).wait()

            pltpu.make_async_copy(
                dsti.at[pl.ds(base, CH)], didx.at[b], semi).wait()

            @pl.when(j + 1 < NCH)
            def _():
                e1 = base + (j + 1) * CH
                pltpu.async_copy(dsti.at[pl.ds(e1, CH)], didx.at[1 - b], semi)

            pltpu.async_copy(ones_v, deg_sh.at[didx.at[b]], sems, add=True)
            return carry

        lax.fori_loop(0, NCH, step, 0)
        pltpu.make_async_copy(ones_v, deg_sh.at[pl.ds(0, CH)], sems).wait()
        pltpu.make_async_copy(ones_v, deg_sh.at[pl.ds(0, CH)], sems).wait()
        plsc.subcore_barrier()

        def pub_chunk(k, c):
            m = sid * 8 + k

            @pl.when(m < NRCH)
            def _():
                r0 = m * CH
                pltpu.sync_copy(deg_sh.at[pl.ds(r0, CH)], ones_v)
                pltpu.sync_copy(ones_v, outdeg.at[cid, pl.ds(r0, CH)])

            return c

        lax.fori_loop(0, 8, pub_chunk, 0)

    return sc_deg


def _tc_in2(x, Wl, Wr, b):
    def body(x_ref, wl_ref, wr_ref, b_ref, yl_ref, yr_ref):
        xv = x_ref[...]
        yl_ref[...] = jnp.dot(xv, wl_ref[...], preferred_element_type=jnp.float32)
        yr_ref[...] = jnp.dot(xv, wr_ref[...], preferred_element_type=jnp.float32) + b_ref[...]

    return pl.pallas_call(
        body,
        out_shape=(jax.ShapeDtypeStruct((N, D), jnp.float32),
                   jax.ShapeDtypeStruct((N, D), jnp.float32)),
    )(x, Wl, Wr, b)


def _tc_mid(p, degp, y1r, Wl2, Wr2, b2):
    blk = 1000

    def body(p_ref, dp_ref, y1r_ref, wl_ref, wr_ref, b_ref, yl_ref, yr_ref):
        dp = dp_ref[...]
        rinv = 1.0 / jnp.maximum(dp[0, :, 0:1] + dp[1, :, 0:1], 1.0)
        pv = p_ref[...]
        h = jnp.maximum((pv[0] + pv[1]) * rinv + y1r_ref[...], 0.0)
        yl_ref[...] = jnp.dot(h, wl_ref[...], preferred_element_type=jnp.float32)
        yr_ref[...] = jnp.dot(h, wr_ref[...], preferred_element_type=jnp.float32) + b_ref[...]

    return pl.pallas_call(
        body,
        grid=(N // blk,),
        in_specs=[
            pl.BlockSpec((NC, blk, D), lambda i: (0, i, 0)),
            pl.BlockSpec((NC, blk, DW), lambda i: (0, i, 0)),
            pl.BlockSpec((blk, D), lambda i: (i, 0)),
            pl.BlockSpec((D, D), lambda i: (0, 0)),
            pl.BlockSpec((D, D), lambda i: (0, 0)),
            pl.BlockSpec((1, D), lambda i: (0, 0)),
        ],
        out_specs=(pl.BlockSpec((blk, D), lambda i: (i, 0)),
                   pl.BlockSpec((blk, D), lambda i: (i, 0))),
        out_shape=(jax.ShapeDtypeStruct((N, D), jnp.float32),
                   jax.ShapeDtypeStruct((N, D), jnp.float32)),
    )(p, degp, y1r, Wl2, Wr2, b2)


def _tc_pool(p, degp, y2r, batch2):
    def body(p_ref, dp_ref, y2r_ref, b_ref, out_ref):
        dp = dp_ref[...]
        rinv = 1.0 / jnp.maximum(dp[0, :, 0:1] + dp[1, :, 0:1], 1.0)
        pv = p_ref[...]
        nodes = (pv[0] + pv[1]) * rinv + y2r_ref[...]
        gids = lax.broadcasted_iota(jnp.int32, (G, N), 0)
        m = (b_ref[...] == gids).astype(jnp.float32)
        s = jnp.dot(m, nodes, preferred_element_type=jnp.float32)
        cnt = jnp.sum(m, axis=1, keepdims=True)
        out_ref[...] = s / jnp.maximum(cnt, 1.0)

    return pl.pallas_call(
        body,
        out_shape=jax.ShapeDtypeStruct((G, D), jnp.float32),
    )(p, degp, y2r, batch2)


def kernel(x, edge_index, edge_attr, batch, Wl1, bl1, Wr1, Wl2, bl2, Wr2):
    src = edge_index[0]
    dst = edge_index[1]
    z2 = jnp.zeros((CH, D), jnp.float32)
    z1 = jnp.zeros((CH, DW), jnp.float32)
    o1 = jnp.ones((CH, DW), jnp.float32)
    (degp,) = _make_sc_deg()(dst, z1, o1)
    y1l, y1r = _tc_in2(x, Wl1, Wr1, bl1.reshape(1, D))
    (p1,) = _make_sc_agg()(y1l, src, dst, z2)
    y2l, y2r = _tc_mid(p1, degp, y1r, Wl2, Wr2, bl2.reshape(1, D))
    (p2,) = _make_sc_agg()(y2l, src, dst, z2)
    return _tc_pool(p2, degp, y2r, batch.reshape(1, N))


# speedup vs baseline: 11.6586x; 1.3436x over previous
import functools

import jax
import jax.numpy as jnp
from jax import lax
from jax.experimental import pallas as pl
from jax.experimental.pallas import tpu as pltpu
from jax.experimental.pallas import tpu_sc as plsc

N = 10000
E = 320000
D = 128
G = 64
DW = 128
NC = 2
NS = 16
NW = NC * NS
CH = 80
EW = E // NW
NCH = EW // CH
NRCH = N // CH


@functools.lru_cache(maxsize=None)
def _make_sc_agg():
    mesh = plsc.VectorSubcoreMesh(core_axis_name="c", subcore_axis_name="s")

    @functools.partial(
        pl.kernel, mesh=mesh,
        out_type=[jax.ShapeDtypeStruct((NC, N, D), jnp.float32)],
        scratch_types=[
            pltpu.VMEM((4, CH), jnp.int32),
            pltpu.VMEM((4, CH), jnp.int32),
            pltpu.VMEM((3, CH, D), jnp.float32),
            pltpu.VMEM_SHARED((N, D), jnp.float32),
            pltpu.SemaphoreType.DMA,
            pltpu.SemaphoreType.DMA,
            pltpu.SemaphoreType.DMA,
        ],
    )
    def sc_agg(table, srci, dsti, z2, outp,
               sidx, didx, rows, acc, semi, semg, sems):
        cid = lax.axis_index("c")
        sid = lax.axis_index("s")
        wid = sid * NC + cid

        pltpu.sync_copy(z2, rows.at[0])

        def zero_chunk(k, c):
            m = sid * 8 + k

            @pl.when(m < NRCH)
            def _():
                pltpu.sync_copy(rows.at[0], acc.at[pl.ds(m * CH, CH)])

            return c

        lax.fori_loop(0, 8, zero_chunk, 0)
        plsc.subcore_barrier()

        base = wid * EW

        pltpu.async_copy(srci.at[pl.ds(base, CH)], sidx.at[0], semi)
        pltpu.async_copy(dsti.at[pl.ds(base, CH)], didx.at[0], semi)
        pltpu.async_copy(srci.at[pl.ds(base + CH, CH)], sidx.at[1], semi)
        pltpu.async_copy(dsti.at[pl.ds(base + CH, CH)], didx.at[1], semi)
        pltpu.make_async_copy(
            srci.at[pl.ds(base, CH)], sidx.at[0], semi).wait()
        pltpu.make_async_copy(
            dsti.at[pl.ds(base, CH)], didx.at[0], semi).wait()
        pltpu.async_copy(table.at[sidx.at[0]], rows.at[0], semg)

        def step(j, carry):
            rb = lax.rem(j, 3)
            rn = lax.rem(j + 1, 3)
            ib = lax.rem(j, 4)
            inx = lax.rem(j + 1, 4)
            inx2 = lax.rem(j + 2, 4)

            @pl.when(j >= 2)
            def _():
                pltpu.make_async_copy(
                    rows.at[rn], acc.at[pl.ds(0, CH)], sems).wait()

            @pl.when(j + 1 < NCH)
            def _():
                pltpu.make_async_copy(
                    srci.at[pl.ds(base, CH)], sidx.at[inx], semi).wait()
                pltpu.make_async_copy(
                    dsti.at[pl.ds(base, CH)], didx.at[inx], semi).wait()
                pltpu.async_copy(table.at[sidx.at[inx]], rows.at[rn], semg)

            @pl.when(j + 2 < NCH)
            def _():
                e2 = base + (j + 2) * CH
                pltpu.async_copy(srci.at[pl.ds(e2, CH)], sidx.at[inx2], semi)
                pltpu.async_copy(dsti.at[pl.ds(e2, CH)], didx.at[inx2], semi)

            pltpu.make_async_copy(
                table.at[sidx.at[ib]], rows.at[rb], semg).wait()
            pltpu.async_copy(rows.at[rb], acc.at[didx.at[ib]], sems, add=True)
            return carry

        lax.fori_loop(0, NCH, step, 0)
        pltpu.make_async_copy(
            rows.at[(NCH - 2) % 3], acc.at[pl.ds(0, CH)], sems).wait()
        pltpu.make_async_copy(
            rows.at[(NCH - 1) % 3], acc.at[pl.ds(0, CH)], sems).wait()
        plsc.subcore_barrier()

        def pub_chunk(k, c):
            m = sid * 8 + k

            @pl.when(m < NRCH)
            def _():
                r0 = m * CH
                pltpu.sync_copy(acc.at[pl.ds(r0, CH)], rows.at[0])
                pltpu.sync_copy(rows.at[0], outp.at[cid, pl.ds(r0, CH)])

            return c

        lax.fori_loop(0, 8, pub_chunk, 0)

    return sc_agg


@functools.lru_cache(maxsize=None)
def _make_sc_deg():
    mesh = plsc.VectorSubcoreMesh(core_axis_name="c", subcore_axis_name="s")

    @functools.partial(
        pl.kernel, mesh=mesh,
        out_type=[jax.ShapeDtypeStruct((NC, N, DW), jnp.float32)],
        scratch_types=[
            pltpu.VMEM((2, CH), jnp.int32),
            pltpu.VMEM((CH, DW), jnp.float32),
            pltpu.VMEM_SHARED((N, DW), jnp.float32),
            pltpu.SemaphoreType.DMA,
            pltpu.SemaphoreType.DMA,
        ],
    )
    def sc_deg(dsti, z1, o1, outdeg, didx, ones_v, deg_sh, semi, sems):
        cid = lax.axis_index("c")
        sid = lax.axis_index("s")
        wid = sid * NC + cid

        pltpu.sync_copy(z1, ones_v)

        def zero_chunk(k, c):
            m = sid * 8 + k

            @pl.when(m < NRCH)
            def _():
                pltpu.sync_copy(ones_v, deg_sh.at[pl.ds(m * CH, CH)])

            return c

        lax.fori_loop(0, 8, zero_chunk, 0)
        pltpu.sync_copy(o1, ones_v)
        plsc.subcore_barrier()

        base = wid * EW

        pltpu.async_copy(dsti.at[pl.ds(base, CH)], didx.at[0], semi)

        def step(j, carry):
            b = lax.rem(j, 2)

            @pl.when(j >= 2)
            def _():
                pltpu.make_async_copy(
                    ones_v, deg_sh.at[pl.ds(0, CH)], sems).wait()

            pltpu.make_async_copy(
                dsti.at[pl.ds(base, CH)], didx.at[b], semi).wait()

            @pl.when(j + 1 < NCH)
            def _():
                e1 = base + (j + 1) * CH
                pltpu.async_copy(dsti.at[pl.ds(e1, CH)], didx.at[1 - b], semi)

            pltpu.async_copy(ones_v, deg_sh.at[didx.at[b]], sems, add=True)
            return carry

        lax.fori_loop(0, NCH, step, 0)
        pltpu.make_async_copy(ones_v, deg_sh.at[pl.ds(0, CH)], sems).wait()
        pltpu.make_async_copy(ones_v, deg_sh.at[pl.ds(0, CH)], sems).wait()
        plsc.subcore_barrier()

        def pub_chunk(k, c):
            m = sid * 8 + k

            @pl.when(m < NRCH)
            def _():
                r0 = m * CH
                pltpu.sync_copy(deg_sh.at[pl.ds(r0, CH)], ones_v)
                pltpu.sync_copy(ones_v, outdeg.at[cid, pl.ds(r0, CH)])

            return c

        lax.fori_loop(0, 8, pub_chunk, 0)

    return sc_deg


def _tc_in2(x, Wl, Wr, b):
    def body(x_ref, wl_ref, wr_ref, b_ref, yl_ref, yr_ref):
        xv = x_ref[...]
        yl_ref[...] = jnp.dot(xv, wl_ref[...], preferred_element_type=jnp.float32)
        yr_ref[...] = jnp.dot(xv, wr_ref[...], preferred_element_type=jnp.float32) + b_ref[...]

    return pl.pallas_call(
        body,
        out_shape=(jax.ShapeDtypeStruct((N, D), jnp.float32),
                   jax.ShapeDtypeStruct((N, D), jnp.float32)),
    )(x, Wl, Wr, b)


def _tc_mid(p, degp, y1r, Wl2, Wr2, b2):
    blk = 1000

    def body(p_ref, dp_ref, y1r_ref, wl_ref, wr_ref, b_ref, yl_ref, yr_ref):
        dp = dp_ref[...]
        rinv = 1.0 / jnp.maximum(dp[0, :, 0:1] + dp[1, :, 0:1], 1.0)
        pv = p_ref[...]
        h = jnp.maximum((pv[0] + pv[1]) * rinv + y1r_ref[...], 0.0)
        yl_ref[...] = jnp.dot(h, wl_ref[...], preferred_element_type=jnp.float32)
        yr_ref[...] = jnp.dot(h, wr_ref[...], preferred_element_type=jnp.float32) + b_ref[...]

    return pl.pallas_call(
        body,
        grid=(N // blk,),
        in_specs=[
            pl.BlockSpec((NC, blk, D), lambda i: (0, i, 0)),
            pl.BlockSpec((NC, blk, DW), lambda i: (0, i, 0)),
            pl.BlockSpec((blk, D), lambda i: (i, 0)),
            pl.BlockSpec((D, D), lambda i: (0, 0)),
            pl.BlockSpec((D, D), lambda i: (0, 0)),
            pl.BlockSpec((1, D), lambda i: (0, 0)),
        ],
        out_specs=(pl.BlockSpec((blk, D), lambda i: (i, 0)),
                   pl.BlockSpec((blk, D), lambda i: (i, 0))),
        out_shape=(jax.ShapeDtypeStruct((N, D), jnp.float32),
                   jax.ShapeDtypeStruct((N, D), jnp.float32)),
    )(p, degp, y1r, Wl2, Wr2, b2)


def _tc_pool(p, degp, y2r, batch2):
    def body(p_ref, dp_ref, y2r_ref, b_ref, out_ref):
        dp = dp_ref[...]
        rinv = 1.0 / jnp.maximum(dp[0, :, 0:1] + dp[1, :, 0:1], 1.0)
        pv = p_ref[...]
        nodes = (pv[0] + pv[1]) * rinv + y2r_ref[...]
        gids = lax.broadcasted_iota(jnp.int32, (G, N), 0)
        m = (b_ref[...] == gids).astype(jnp.float32)
        s = jnp.dot(m, nodes, preferred_element_type=jnp.float32)
        cnt = jnp.sum(m, axis=1, keepdims=True)
        out_ref[...] = s / jnp.maximum(cnt, 1.0)

    return pl.pallas_call(
        body,
        out_shape=jax.ShapeDtypeStruct((G, D), jnp.float32),
    )(p, degp, y2r, batch2)


def kernel(x, edge_index, edge_attr, batch, Wl1, bl1, Wr1, Wl2, bl2, Wr2):
    src = edge_index[0]
    dst = edge_index[1]
    z2 = jnp.zeros((CH, D), jnp.float32)
    z1 = jnp.zeros((CH, DW), jnp.float32)
    o1 = jnp.ones((CH, DW), jnp.float32)
    (degp,) = _make_sc_deg()(dst, z1, o1)
    y1l, y1r = _tc_in2(x, Wl1, Wr1, bl1.reshape(1, D))
    (p1,) = _make_sc_agg()(y1l, src, dst, z2)
    y2l, y2r = _tc_mid(p1, degp, y1r, Wl2, Wr2, bl2.reshape(1, D))
    (p2,) = _make_sc_agg()(y2l, src, dst, z2)
    return _tc_pool(p2, degp, y2r, batch.reshape(1, N))


# speedup vs baseline: 11.9184x; 1.0223x over previous
import functools

import jax
import jax.numpy as jnp
from jax import lax
from jax.experimental import pallas as pl
from jax.experimental.pallas import tpu as pltpu
from jax.experimental.pallas import tpu_sc as plsc

N = 10000
E = 320000
D = 128
G = 64
DW = 128
NC = 2
NS = 16
NW = NC * NS
CH = 80
EW = E // NW
NCH = EW // CH
NRCH = N // CH


@functools.lru_cache(maxsize=None)
def _make_sc_agg():
    mesh = plsc.VectorSubcoreMesh(core_axis_name="c", subcore_axis_name="s")

    @functools.partial(
        pl.kernel, mesh=mesh,
        out_type=[jax.ShapeDtypeStruct((NC, N, D), jnp.float32)],
        scratch_types=[
            pltpu.VMEM((4, CH), jnp.int32),
            pltpu.VMEM((4, CH), jnp.int32),
            pltpu.VMEM((3, CH, D), jnp.float32),
            pltpu.VMEM_SHARED((N, D), jnp.float32),
            pltpu.SemaphoreType.DMA,
            pltpu.SemaphoreType.DMA,
            pltpu.SemaphoreType.DMA,
        ],
    )
    def sc_agg(table, srci, dsti, z2, outp,
               sidx, didx, rows, acc, semi, semg, sems):
        cid = lax.axis_index("c")
        sid = lax.axis_index("s")
        wid = sid * NC + cid

        pltpu.sync_copy(z2, rows.at[0])

        def zero_chunk(k, c):
            m = sid * 8 + k

            @pl.when(m < NRCH)
            def _():
                pltpu.sync_copy(rows.at[0], acc.at[pl.ds(m * CH, CH)])

            return c

        lax.fori_loop(0, 8, zero_chunk, 0)

        base = wid * EW

        pltpu.async_copy(srci.at[pl.ds(base, CH)], sidx.at[0], semi)
        pltpu.async_copy(dsti.at[pl.ds(base, CH)], didx.at[0], semi)
        pltpu.async_copy(srci.at[pl.ds(base + CH, CH)], sidx.at[1], semi)
        pltpu.async_copy(dsti.at[pl.ds(base + CH, CH)], didx.at[1], semi)
        pltpu.make_async_copy(
            srci.at[pl.ds(base, CH)], sidx.at[0], semi).wait()
        pltpu.make_async_copy(
            dsti.at[pl.ds(base, CH)], didx.at[0], semi).wait()
        pltpu.async_copy(table.at[sidx.at[0]], rows.at[0], semg)
        plsc.subcore_barrier()

        def step(j, carry):
            rb = lax.rem(j, 3)
            rn = lax.rem(j + 1, 3)
            ib = lax.rem(j, 4)
            inx = lax.rem(j + 1, 4)
            inx2 = lax.rem(j + 2, 4)

            @pl.when(j >= 2)
            def _():
                pltpu.make_async_copy(
                    rows.at[rn], acc.at[pl.ds(0, CH)], sems).wait()

            @pl.when(j + 1 < NCH)
            def _():
                pltpu.make_async_copy(
                    srci.at[pl.ds(base, CH)], sidx.at[inx], semi).wait()
                pltpu.make_async_copy(
                    dsti.at[pl.ds(base, CH)], didx.at[inx], semi).wait()
                pltpu.async_copy(table.at[sidx.at[inx]], rows.at[rn], semg)

            @pl.when(j + 2 < NCH)
            def _():
                e2 = base + (j + 2) * CH
                pltpu.async_copy(srci.at[pl.ds(e2, CH)], sidx.at[inx2], semi)
                pltpu.async_copy(dsti.at[pl.ds(e2, CH)], didx.at[inx2], semi)

            pltpu.make_async_copy(
                table.at[sidx.at[ib]], rows.at[rb], semg).wait()
            pltpu.async_copy(rows.at[rb], acc.at[didx.at[ib]], sems, add=True)
            return carry

        lax.fori_loop(0, NCH, step, 0)
        pltpu.make_async_copy(
            rows.at[(NCH - 2) % 3], acc.at[pl.ds(0, CH)], sems).wait()
        pltpu.make_async_copy(
            rows.at[(NCH - 1) % 3], acc.at[pl.ds(0, CH)], sems).wait()
        plsc.subcore_barrier()

        cnt = NRCH - sid * 8

        def pub_chunk(k, c):
            m = sid * 8 + k
            b = lax.rem(k, 2)

            @pl.when(jnp.logical_and(k >= 2, k - 2 < cnt))
            def _():
                pltpu.make_async_copy(
                    rows.at[b], outp.at[cid, pl.ds(0, CH)], sems).wait()

            @pl.when(m < NRCH)
            def _():
                r0 = m * CH
                pltpu.sync_copy(acc.at[pl.ds(r0, CH)], rows.at[b])
                pltpu.async_copy(rows.at[b], outp.at[cid, pl.ds(r0, CH)], sems)

            return c

        lax.fori_loop(0, 8, pub_chunk, 0)

        @pl.when(cnt >= 8)
        def _():
            pltpu.make_async_copy(
                rows.at[0], outp.at[cid, pl.ds(0, CH)], sems).wait()
            pltpu.make_async_copy(
                rows.at[1], outp.at[cid, pl.ds(0, CH)], sems).wait()

    return sc_agg


@functools.lru_cache(maxsize=None)
def _make_sc_deg():
    mesh = plsc.VectorSubcoreMesh(core_axis_name="c", subcore_axis_name="s")

    @functools.partial(
        pl.kernel, mesh=mesh,
        out_type=[jax.ShapeDtypeStruct((NC, N, DW), jnp.float32)],
        scratch_types=[
            pltpu.VMEM((2, CH), jnp.int32),
            pltpu.VMEM((CH, DW), jnp.float32),
            pltpu.VMEM((2, CH, DW), jnp.float32),
            pltpu.VMEM_SHARED((N, DW), jnp.float32),
            pltpu.SemaphoreType.DMA,
            pltpu.SemaphoreType.DMA,
        ],
    )
    def sc_deg(dsti, z1, o1, outdeg, didx, ones_v, dstage, deg_sh, semi, sems):
        cid = lax.axis_index("c")
        sid = lax.axis_index("s")
        wid = sid * NC + cid

        pltpu.sync_copy(z1, ones_v)

        def zero_chunk(k, c):
            m = sid * 8 + k

            @pl.when(m < NRCH)
            def _():
                pltpu.sync_copy(ones_v, deg_sh.at[pl.ds(m * CH, CH)])

            return c

        lax.fori_loop(0, 8, zero_chunk, 0)
        pltpu.sync_copy(o1, ones_v)

        base = wid * EW

        pltpu.async_copy(dsti.at[pl.ds(base, CH)], didx.at[0], semi)
        plsc.subcore_barrier()

        def step(j, carry):
            b = lax.rem(j, 2)

            @pl.when(j >= 2)
            def _():
                pltpu.make_async_copy(
                    ones_v, deg_sh.at[pl.ds(0, CH)], sems).wait()

            pltpu.make_async_copy(
                dsti.at[pl.ds(base, CH)], didx.at[b], semi).wait()

            @pl.when(j + 1 < NCH)
            def _():
                e1 = base + (j + 1) * CH
                pltpu.async_copy(dsti.at[pl.ds(e1, CH)], didx.at[1 - b], semi)

            pltpu.async_copy(ones_v, deg_sh.at[didx.at[b]], sems, add=True)
            return carry

        lax.fori_loop(0, NCH, step, 0)
        pltpu.make_async_copy(ones_v, deg_sh.at[pl.ds(0, CH)], sems).wait()
        pltpu.make_async_copy(ones_v, deg_sh.at[pl.ds(0, CH)], sems).wait()
        plsc.subcore_barrier()

        cnt = NRCH - sid * 8

        def pub_chunk(k, c):
            m = sid * 8 + k
            b = lax.rem(k, 2)

            @pl.when(jnp.logical_and(k >= 2, k - 2 < cnt))
            def _():
                pltpu.make_async_copy(
                    dstage.at[b], outdeg.at[cid, pl.ds(0, CH)], sems).wait()

            @pl.when(m < NRCH)
            def _():
                r0 = m * CH
                pltpu.sync_copy(deg_sh.at[pl.ds(r0, CH)], dstage.at[b])
                pltpu.async_copy(
                    dstage.at[b], outdeg.at[cid, pl.ds(r0, CH)], sems)

            return c

        lax.fori_loop(0, 8, pub_chunk, 0)

        @pl.when(cnt >= 8)
        def _():
            pltpu.make_async_copy(
                dstage.at[0], outdeg.at[cid, pl.ds(0, CH)], sems).wait()
            pltpu.make_async_copy(
                dstage.at[1], outdeg.at[cid, pl.ds(0, CH)], sems).wait()

    return sc_deg


def _tc_in2(x, Wl, Wr, b):
    def body(x_ref, wl_ref, wr_ref, b_ref, yl_ref, yr_ref):
        xv = x_ref[...]
        yl_ref[...] = jnp.dot(xv, wl_ref[...], preferred_element_type=jnp.float32)
        yr_ref[...] = jnp.dot(xv, wr_ref[...], preferred_element_type=jnp.float32) + b_ref[...]

    return pl.pallas_call(
        body,
        out_shape=(jax.ShapeDtypeStruct((N, D), jnp.float32),
                   jax.ShapeDtypeStruct((N, D), jnp.float32)),
    )(x, Wl, Wr, b)


def _tc_mid(p, degp, y1r, Wl2, Wr2, b2):
    blk = 1000

    def body(p_ref, dp_ref, y1r_ref, wl_ref, wr_ref, b_ref, yl_ref, yr_ref):
        dp = dp_ref[...]
        rinv = 1.0 / jnp.maximum(dp[0, :, 0:1] + dp[1, :, 0:1], 1.0)
        pv = p_ref[...]
        h = jnp.maximum((pv[0] + pv[1]) * rinv + y1r_ref[...], 0.0)
        yl_ref[...] = jnp.dot(h, wl_ref[...], preferred_element_type=jnp.float32)
        yr_ref[...] = jnp.dot(h, wr_ref[...], preferred_element_type=jnp.float32) + b_ref[...]

    return pl.pallas_call(
        body,
        grid=(N // blk,),
        in_specs=[
            pl.BlockSpec((NC, blk, D), lambda i: (0, i, 0)),
            pl.BlockSpec((NC, blk, DW), lambda i: (0, i, 0)),
            pl.BlockSpec((blk, D), lambda i: (i, 0)),
            pl.BlockSpec((D, D), lambda i: (0, 0)),
            pl.BlockSpec((D, D), lambda i: (0, 0)),
            pl.BlockSpec((1, D), lambda i: (0, 0)),
        ],
        out_specs=(pl.BlockSpec((blk, D), lambda i: (i, 0)),
                   pl.BlockSpec((blk, D), lambda i: (i, 0))),
        out_shape=(jax.ShapeDtypeStruct((N, D), jnp.float32),
                   jax.ShapeDtypeStruct((N, D), jnp.float32)),
    )(p, degp, y1r, Wl2, Wr2, b2)


def _tc_pool(p, degp, y2r, batch2):
    def body(p_ref, dp_ref, y2r_ref, b_ref, out_ref):
        dp = dp_ref[...]
        rinv = 1.0 / jnp.maximum(dp[0, :, 0:1] + dp[1, :, 0:1], 1.0)
        pv = p_ref[...]
        nodes = (pv[0] + pv[1]) * rinv + y2r_ref[...]
        gids = lax.broadcasted_iota(jnp.int32, (G, N), 0)
        m = (b_ref[...] == gids).astype(jnp.float32)
        s = jnp.dot(m, nodes, preferred_element_type=jnp.float32)
        cnt = jnp.sum(m, axis=1, keepdims=True)
        out_ref[...] = s / jnp.maximum(cnt, 1.0)

    return pl.pallas_call(
        body,
        out_shape=jax.ShapeDtypeStruct((G, D), jnp.float32),
    )(p, degp, y2r, batch2)


def kernel(x, edge_index, edge_attr, batch, Wl1, bl1, Wr1, Wl2, bl2, Wr2):
    src = edge_index[0]
    dst = edge_index[1]
    z2 = jnp.zeros((CH, D), jnp.float32)
    z1 = jnp.zeros((CH, DW), jnp.float32)
    o1 = jnp.ones((CH, DW), jnp.float32)
    (degp,) = _make_sc_deg()(dst, z1, o1)
    y1l, y1r = _tc_in2(x, Wl1, Wr1, bl1.reshape(1, D))
    (p1,) = _make_sc_agg()(y1l, src, dst, z2)
    y2l, y2r = _tc_mid(p1, degp, y1r, Wl2, Wr2, bl2.reshape(1, D))
    (p2,) = _make_sc_agg()(y2l, src, dst, z2)
    return _tc_pool(p2, degp, y2r, batch.reshape(1, N))


# speedup vs baseline: 12.0993x; 1.0152x over previous
import functools

import jax
import jax.numpy as jnp
from jax import lax
from jax.experimental import pallas as pl
from jax.experimental.pallas import tpu as pltpu
from jax.experimental.pallas import tpu_sc as plsc

N = 10000
E = 320000
D = 128
G = 64
DW = 128
NC = 2
NS = 16
NW = NC * NS
CH = 80
EW = E // NW
CHL = 128
NCHL = EW // CHL
TAIL = EW - NCHL * CHL
NCH = EW // CH
NRCH = N // CH


@functools.lru_cache(maxsize=None)
def _make_sc_agg():
    mesh = plsc.VectorSubcoreMesh(core_axis_name="c", subcore_axis_name="s")

    @functools.partial(
        pl.kernel, mesh=mesh,
        out_type=[jax.ShapeDtypeStruct((NC, N, D), jnp.float32)],
        scratch_types=[
            pltpu.VMEM((4, CHL), jnp.int32),
            pltpu.VMEM((4, CHL), jnp.int32),
            pltpu.VMEM((16,), jnp.int32),
            pltpu.VMEM((16,), jnp.int32),
            pltpu.VMEM((3, CHL, D), jnp.float32),
            pltpu.VMEM_SHARED((N, D), jnp.float32),
            pltpu.SemaphoreType.DMA,
            pltpu.SemaphoreType.DMA,
            pltpu.SemaphoreType.DMA,
        ],
    )
    def sc_agg(table, srci, dsti, z2, outp,
               sidx, didx, sidx_t, didx_t, rows, acc, semi, semg, sems):
        cid = lax.axis_index("c")
        sid = lax.axis_index("s")
        wid = sid * NC + cid

        pltpu.sync_copy(z2, rows.at[0, pl.ds(0, CH)])

        def zero_chunk(k, c):
            m = sid * 8 + k

            @pl.when(m < NRCH)
            def _():
                pltpu.sync_copy(rows.at[0, pl.ds(0, CH)],
                                acc.at[pl.ds(m * CH, CH)])

            return c

        lax.fori_loop(0, 8, zero_chunk, 0)

        base = wid * EW

        pltpu.async_copy(srci.at[pl.ds(base, CHL)], sidx.at[0], semi)
        pltpu.async_copy(dsti.at[pl.ds(base, CHL)], didx.at[0], semi)
        pltpu.async_copy(srci.at[pl.ds(base + CHL, CHL)], sidx.at[1], semi)
        pltpu.async_copy(dsti.at[pl.ds(base + CHL, CHL)], didx.at[1], semi)
        pltpu.make_async_copy(
            srci.at[pl.ds(base, CHL)], sidx.at[0], semi).wait()
        pltpu.make_async_copy(
            dsti.at[pl.ds(base, CHL)], didx.at[0], semi).wait()
        pltpu.async_copy(table.at[sidx.at[0]], rows.at[0], semg)
        plsc.subcore_barrier()

        def step(j, carry):
            rb = lax.rem(j, 3)
            rn = lax.rem(j + 1, 3)
            ib = lax.rem(j, 4)
            inx = lax.rem(j + 1, 4)
            inx2 = lax.rem(j + 2, 4)

            @pl.when(j >= 2)
            def _():
                pltpu.make_async_copy(
                    rows.at[rn], acc.at[pl.ds(0, CHL)], sems).wait()

            @pl.when(j + 1 < NCHL)
            def _():
                pltpu.make_async_copy(
                    srci.at[pl.ds(base, CHL)], sidx.at[inx], semi).wait()
                pltpu.make_async_copy(
                    dsti.at[pl.ds(base, CHL)], didx.at[inx], semi).wait()
                pltpu.async_copy(table.at[sidx.at[inx]], rows.at[rn], semg)

            @pl.when(j + 2 < NCHL)
            def _():
                e2 = base + (j + 2) * CHL
                pltpu.async_copy(srci.at[pl.ds(e2, CHL)], sidx.at[inx2], semi)
                pltpu.async_copy(dsti.at[pl.ds(e2, CHL)], didx.at[inx2], semi)

            pltpu.make_async_copy(
                table.at[sidx.at[ib]], rows.at[rb], semg).wait()
            pltpu.async_copy(rows.at[rb], acc.at[didx.at[ib]], sems, add=True)
            return carry

        lax.fori_loop(0, NCHL, step, 0)
        pltpu.make_async_copy(
            rows.at[(NCHL - 2) % 3], acc.at[pl.ds(0, CHL)], sems).wait()
        pltpu.make_async_copy(
            rows.at[(NCHL - 1) % 3], acc.at[pl.ds(0, CHL)], sems).wait()

        et = base + NCHL * CHL
        pltpu.sync_copy(srci.at[pl.ds(et, TAIL)], sidx_t)
        pltpu.sync_copy(dsti.at[pl.ds(et, TAIL)], didx_t)
        pltpu.async_copy(
            table.at[sidx_t], rows.at[0, pl.ds(0, TAIL)], semg).wait()
        pltpu.sync_copy(rows.at[0, pl.ds(0, TAIL)], acc.at[didx_t], add=True)
        plsc.subcore_barrier()

        cnt = NRCH - sid * 8

        def pub_chunk(k, c):
            m = sid * 8 + k
            b = lax.rem(k, 2)

            @pl.when(jnp.logical_and(k >= 2, k - 2 < cnt))
            def _():
                pltpu.make_async_copy(
                    rows.at[b, pl.ds(0, CH)],
                    outp.at[cid, pl.ds(0, CH)], sems).wait()

            @pl.when(m < NRCH)
            def _():
                r0 = m * CH
                pltpu.sync_copy(acc.at[pl.ds(r0, CH)], rows.at[b, pl.ds(0, CH)])
                pltpu.async_copy(rows.at[b, pl.ds(0, CH)],
                                 outp.at[cid, pl.ds(r0, CH)], sems)

            return c

        lax.fori_loop(0, 8, pub_chunk, 0)

        @pl.when(cnt >= 8)
        def _():
            pltpu.make_async_copy(
                rows.at[0, pl.ds(0, CH)],
                outp.at[cid, pl.ds(0, CH)], sems).wait()
            pltpu.make_async_copy(
                rows.at[1, pl.ds(0, CH)],
                outp.at[cid, pl.ds(0, CH)], sems).wait()

    return sc_agg


@functools.lru_cache(maxsize=None)
def _make_sc_deg():
    mesh = plsc.VectorSubcoreMesh(core_axis_name="c", subcore_axis_name="s")

    @functools.partial(
        pl.kernel, mesh=mesh,
        out_type=[jax.ShapeDtypeStruct((NC, N, DW), jnp.float32)],
        scratch_types=[
            pltpu.VMEM((2, CH), jnp.int32),
            pltpu.VMEM((CH, DW), jnp.float32),
            pltpu.VMEM((2, CH, DW), jnp.float32),
            pltpu.VMEM_SHARED((N, DW), jnp.float32),
            pltpu.SemaphoreType.DMA,
            pltpu.SemaphoreType.DMA,
        ],
    )
    def sc_deg(dsti, z1, o1, outdeg, didx, ones_v, dstage, deg_sh, semi, sems):
        cid = lax.axis_index("c")
        sid = lax.axis_index("s")
        wid = sid * NC + cid

        pltpu.sync_copy(z1, ones_v)

        def zero_chunk(k, c):
            m = sid * 8 + k

            @pl.when(m < NRCH)
            def _():
                pltpu.sync_copy(ones_v, deg_sh.at[pl.ds(m * CH, CH)])

            return c

        lax.fori_loop(0, 8, zero_chunk, 0)
        pltpu.sync_copy(o1, ones_v)

        base = wid * EW

        pltpu.async_copy(dsti.at[pl.ds(base, CH)], didx.at[0], semi)
        plsc.subcore_barrier()

        def step(j, carry):
            b = lax.rem(j, 2)

            @pl.when(j >= 2)
            def _():
                pltpu.make_async_copy(
                    ones_v, deg_sh.at[pl.ds(0, CH)], sems).wait()

            pltpu.make_async_copy(
                dsti.at[pl.ds(base, CH)], didx.at[b], semi).wait()

            @pl.when(j + 1 < NCH)
            def _():
                e1 = base + (j + 1) * CH
                pltpu.async_copy(dsti.at[pl.ds(e1, CH)], didx.at[1 - b], semi)

            pltpu.async_copy(ones_v, deg_sh.at[didx.at[b]], sems, add=True)
            return carry

        lax.fori_loop(0, NCH, step, 0)
        pltpu.make_async_copy(ones_v, deg_sh.at[pl.ds(0, CH)], sems).wait()
        pltpu.make_async_copy(ones_v, deg_sh.at[pl.ds(0, CH)], sems).wait()
        plsc.subcore_barrier()

        cnt = NRCH - sid * 8

        def pub_chunk(k, c):
            m = sid * 8 + k
            b = lax.rem(k, 2)

            @pl.when(jnp.logical_and(k >= 2, k - 2 < cnt))
            def _():
                pltpu.make_async_copy(
                    dstage.at[b], outdeg.at[cid, pl.ds(0, CH)], sems).wait()

            @pl.when(m < NRCH)
            def _():
                r0 = m * CH
                pltpu.sync_copy(deg_sh.at[pl.ds(r0, CH)], dstage.at[b])
                pltpu.async_copy(
                    dstage.at[b], outdeg.at[cid, pl.ds(r0, CH)], sems)

            return c

        lax.fori_loop(0, 8, pub_chunk, 0)

        @pl.when(cnt >= 8)
        def _():
            pltpu.make_async_copy(
                dstage.at[0], outdeg.at[cid, pl.ds(0, CH)], sems).wait()
            pltpu.make_async_copy(
                dstage.at[1], outdeg.at[cid, pl.ds(0, CH)], sems).wait()

    return sc_deg


def _tc_in2(x, Wl, Wr, b):
    def body(x_ref, wl_ref, wr_ref, b_ref, yl_ref, yr_ref):
        xv = x_ref[...]
        yl_ref[...] = jnp.dot(xv, wl_ref[...], preferred_element_type=jnp.float32)
        yr_ref[...] = jnp.dot(xv, wr_ref[...], preferred_element_type=jnp.float32) + b_ref[...]

    return pl.pallas_call(
        body,
        out_shape=(jax.ShapeDtypeStruct((N, D), jnp.float32),
                   jax.ShapeDtypeStruct((N, D), jnp.float32)),
    )(x, Wl, Wr, b)


def _tc_mid(p, degp, y1r, Wl2, Wr2, b2):
    blk = 1000

    def body(p_ref, dp_ref, y1r_ref, wl_ref, wr_ref, b_ref, yl_ref, yr_ref):
        dp = dp_ref[...]
        rinv = 1.0 / jnp.maximum(dp[0, :, 0:1] + dp[1, :, 0:1], 1.0)
        pv = p_ref[...]
        h = jnp.maximum((pv[0] + pv[1]) * rinv + y1r_ref[...], 0.0)
        yl_ref[...] = jnp.dot(h, wl_ref[...], preferred_element_type=jnp.float32)
        yr_ref[...] = jnp.dot(h, wr_ref[...], preferred_element_type=jnp.float32) + b_ref[...]

    return pl.pallas_call(
        body,
        grid=(N // blk,),
        in_specs=[
            pl.BlockSpec((NC, blk, D), lambda i: (0, i, 0)),
            pl.BlockSpec((NC, blk, DW), lambda i: (0, i, 0)),
            pl.BlockSpec((blk, D), lambda i: (i, 0)),
            pl.BlockSpec((D, D), lambda i: (0, 0)),
            pl.BlockSpec((D, D), lambda i: (0, 0)),
            pl.BlockSpec((1, D), lambda i: (0, 0)),
        ],
        out_specs=(pl.BlockSpec((blk, D), lambda i: (i, 0)),
                   pl.BlockSpec((blk, D), lambda i: (i, 0))),
        out_shape=(jax.ShapeDtypeStruct((N, D), jnp.float32),
                   jax.ShapeDtypeStruct((N, D), jnp.float32)),
    )(p, degp, y1r, Wl2, Wr2, b2)


def _tc_pool(p, degp, y2r, batch2):
    def body(p_ref, dp_ref, y2r_ref, b_ref, out_ref):
        dp = dp_ref[...]
        rinv = 1.0 / jnp.maximum(dp[0, :, 0:1] + dp[1, :, 0:1], 1.0)
        pv = p_ref[...]
        nodes = (pv[0] + pv[1]) * rinv + y2r_ref[...]
        gids = lax.broadcasted_iota(jnp.int32, (G, N), 0)
        m = (b_ref[...] == gids).astype(jnp.float32)
        s = jnp.dot(m, nodes, preferred_element_type=jnp.float32)
        cnt = jnp.sum(m, axis=1, keepdims=True)
        out_ref[...] = s / jnp.maximum(cnt, 1.0)

    return pl.pallas_call(
        body,
        out_shape=jax.ShapeDtypeStruct((G, D), jnp.float32),
    )(p, degp, y2r, batch2)


def kernel(x, edge_index, edge_attr, batch, Wl1, bl1, Wr1, Wl2, bl2, Wr2):
    src = edge_index[0]
    dst = edge_index[1]
    z2 = jnp.zeros((CH, D), jnp.float32)
    z1 = jnp.zeros((CH, DW), jnp.float32)
    o1 = jnp.ones((CH, DW), jnp.float32)
    (degp,) = _make_sc_deg()(dst, z1, o1)
    y1l, y1r = _tc_in2(x, Wl1, Wr1, bl1.reshape(1, D))
    (p1,) = _make_sc_agg()(y1l, src, dst, z2)
    y2l, y2r = _tc_mid(p1, degp, y1r, Wl2, Wr2, bl2.reshape(1, D))
    (p2,) = _make_sc_agg()(y2l, src, dst, z2)
    return _tc_pool(p2, degp, y2r, batch.reshape(1, N))
